# baseline (speedup 1.0000x reference)
import functools

import jax
import jax.numpy as jnp
from jax import lax
from jax.experimental import pallas as pl
from jax.experimental.pallas import tpu as pltpu
from jax.experimental.pallas import tpu_sc as plsc

_N = 4096
_V = 32000
_LANES = 16
_NW = 32
_RPW = _N // _NW
_CHUNKS = _RPW // _LANES


@functools.partial(
    pl.kernel,
    mesh=plsc.VectorSubcoreMesh(core_axis_name="c", subcore_axis_name="s"),
    out_type=jax.ShapeDtypeStruct((_NW * _LANES,), jnp.float32),
    compiler_params=pltpu.CompilerParams(needs_layout_passes=False),
    scratch_types=[
        pltpu.VMEM((_RPW,), jnp.int32),
        pltpu.VMEM((_LANES, 8, 128), jnp.float32),
        pltpu.VMEM((_LANES,), jnp.float32),
        pltpu.SemaphoreType.DMA,
    ],
)
def _sc_gather(x_hbm, tgt_hbm, out_hbm, tgt_v, tiles_v, acc_v, sem):
    wid = lax.axis_index("s") * 2 + lax.axis_index("c")
    base = wid * _RPW
    pltpu.sync_copy(tgt_hbm.at[pl.ds(base, _RPW)], tgt_v)
    acc = jnp.zeros((_LANES,), jnp.float32)
    for c in range(_CHUNKS):
        tc_vec = tgt_v[pl.ds(c * _LANES, _LANES)]
        copies = []
        for k in range(_LANES):
            t = tc_vec[k]
            col0 = pl.multiple_of(lax.bitwise_and(t, jnp.int32(~127)), 128)
            r0 = base + c * _LANES + 8 * (k // 8)
            copies.append(
                pltpu.async_copy(
                    x_hbm.at[pl.ds(r0, 8), pl.ds(col0, 128)], tiles_v.at[k], sem
                )
            )
        for cp in copies:
            cp.wait()
        t = tgt_v[pl.ds(c * _LANES, _LANES)]
        ii = lax.iota(jnp.int32, _LANES)
        rowin = lax.bitwise_and(ii, 7)
        lane = lax.bitwise_and(t, 127)
        vals = plsc.load_gather(tiles_v, [ii, rowin, lane])
        acc = acc + jnp.where(t != 0, vals, 0.0)
    acc_v[...] = acc
    pltpu.sync_copy(acc_v, out_hbm.at[pl.ds(wid * _LANES, _LANES)])


def _loss_body(tgt_ref, x_ref, out_ref, m_ref, s_ref):
    r = pl.program_id(0)
    c = pl.program_id(1)
    x = x_ref[...]
    t = tgt_ref[...]
    R = x.shape[0]

    @pl.when(c == 0)
    def _():
        m_ref[...] = jnp.full((R, 1), -jnp.inf, jnp.float32)
        s_ref[...] = jnp.zeros((R, 1), jnp.float32)

    bm = jnp.max(x, axis=1, keepdims=True)
    m_old = m_ref[...]
    m_new = jnp.maximum(m_old, bm)
    s_ref[...] = s_ref[...] * jnp.exp(m_old - m_new) + jnp.sum(
        jnp.exp(x - m_new), axis=1, keepdims=True
    )
    m_ref[...] = m_new

    @pl.when(c == pl.num_programs(1) - 1)
    def _():
        lse = jnp.log(s_ref[...]) + m_ref[...]
        part = jnp.sum(jnp.where(t != 0, lse, 0.0), axis=(0, 1), keepdims=True)

        @pl.when(r == 0)
        def _():
            out_ref[...] = part

        @pl.when(r > 0)
        def _():
            out_ref[...] += part


@functools.partial(jax.jit, static_argnames=("row_block", "col_block", "interpret"))
def _loss_call(inp, tgt, row_block=64, col_block=32000, interpret=False):
    n, v = inp.shape
    out = pl.pallas_call(
        _loss_body,
        grid=(n // row_block, v // col_block),
        in_specs=[
            pl.BlockSpec((row_block, 1), lambda r, c: (r, 0)),
            pl.BlockSpec((row_block, col_block), lambda r, c: (r, c)),
        ],
        out_specs=pl.BlockSpec((1, 1), lambda r, c: (0, 0)),
        out_shape=jax.ShapeDtypeStruct((1, 1), jnp.float32),
        scratch_shapes=[
            pltpu.VMEM((row_block, 1), jnp.float32),
            pltpu.VMEM((row_block, 1), jnp.float32),
        ],
        interpret=interpret,
    )(tgt, inp)
    return out[0, 0]


def kernel(inp, target):
    n, v = inp.shape
    tgt = target.astype(jnp.int32)
    sc_part = _sc_gather(inp, tgt)
    lse_sum = _loss_call(inp, tgt.reshape(n, 1))
    return lse_sum - jnp.sum(sc_part)


# speedup vs baseline: 1.0994x; 1.0994x over previous
import functools

import jax
import jax.numpy as jnp
from jax import lax
from jax.experimental import pallas as pl
from jax.experimental.pallas import tpu as pltpu
from jax.experimental.pallas import tpu_sc as plsc

_N = 4096
_V = 32000
_LANES = 16
_NW = 32
_RPW = _N // _NW
_CHUNKS = _RPW // _LANES


@functools.partial(
    pl.kernel,
    mesh=plsc.VectorSubcoreMesh(core_axis_name="c", subcore_axis_name="s"),
    out_type=jax.ShapeDtypeStruct((_NW * _LANES,), jnp.float32),
    compiler_params=pltpu.CompilerParams(needs_layout_passes=False),
    scratch_types=[
        pltpu.VMEM((_RPW,), jnp.int32),
        pltpu.VMEM((_LANES, 8, 128), jnp.float32),
        pltpu.VMEM((_LANES,), jnp.float32),
        pltpu.SemaphoreType.DMA,
    ],
)
def _sc_gather(x_hbm, tgt_hbm, out_hbm, tgt_v, tiles_v, acc_v, sem):
    wid = lax.axis_index("s") * 2 + lax.axis_index("c")
    base = wid * _RPW
    pltpu.sync_copy(tgt_hbm.at[pl.ds(base, _RPW)], tgt_v)
    acc = jnp.zeros((_LANES,), jnp.float32)
    for c in range(_CHUNKS):
        tc_vec = tgt_v[pl.ds(c * _LANES, _LANES)]
        copies = []
        for k in range(_LANES):
            t = tc_vec[k]
            col0 = pl.multiple_of(lax.bitwise_and(t, jnp.int32(~127)), 128)
            r0 = base + c * _LANES + 8 * (k // 8)
            copies.append(
                pltpu.async_copy(
                    x_hbm.at[pl.ds(r0, 8), pl.ds(col0, 128)], tiles_v.at[k], sem
                )
            )
        for cp in copies:
            cp.wait()
        t = tgt_v[pl.ds(c * _LANES, _LANES)]
        ii = lax.iota(jnp.int32, _LANES)
        rowin = lax.bitwise_and(ii, 7)
        lane = lax.bitwise_and(t, 127)
        vals = plsc.load_gather(tiles_v, [ii, rowin, lane])
        acc = acc + jnp.where(t != 0, vals, 0.0)
    acc_v[...] = acc
    pltpu.sync_copy(acc_v, out_hbm.at[pl.ds(wid * _LANES, _LANES)])


def _loss_body(tgt_ref, x_ref, out_ref, m_ref, s_ref):
    r = pl.program_id(0)
    c = pl.program_id(1)
    x = x_ref[...]
    t = tgt_ref[...]
    R = x.shape[0]

    @pl.when(c == 0)
    def _():
        m_ref[...] = jnp.full((R, 1), -jnp.inf, jnp.float32)
        s_ref[...] = jnp.zeros((R, 1), jnp.float32)

    bm = jnp.max(x, axis=1, keepdims=True)
    m_old = m_ref[...]
    m_new = jnp.maximum(m_old, bm)
    s_ref[...] = s_ref[...] * jnp.exp(m_old - m_new) + jnp.sum(
        jnp.exp(x - m_new), axis=1, keepdims=True
    )
    m_ref[...] = m_new

    @pl.when(c == pl.num_programs(1) - 1)
    def _():
        lse = jnp.log(s_ref[...]) + m_ref[...]
        part = jnp.sum(jnp.where(t != 0, lse, 0.0), axis=(0, 1), keepdims=True)

        @pl.when(r == 0)
        def _():
            out_ref[...] = part

        @pl.when(r > 0)
        def _():
            out_ref[...] += part


@functools.partial(jax.jit, static_argnames=("row_block", "col_block", "interpret"))
def _loss_call(inp, tgt, row_block=256, col_block=32000, interpret=False):
    n, v = inp.shape
    out = pl.pallas_call(
        _loss_body,
        grid=(n // row_block, v // col_block),
        in_specs=[
            pl.BlockSpec((row_block, 1), lambda r, c: (r, 0)),
            pl.BlockSpec((row_block, col_block), lambda r, c: (r, c)),
        ],
        out_specs=pl.BlockSpec((1, 1), lambda r, c: (0, 0)),
        out_shape=jax.ShapeDtypeStruct((1, 1), jnp.float32),
        scratch_shapes=[
            pltpu.VMEM((row_block, 1), jnp.float32),
            pltpu.VMEM((row_block, 1), jnp.float32),
        ],
        interpret=interpret,
        compiler_params=pltpu.CompilerParams(vmem_limit_bytes=100 * 1024 * 1024),
    )(tgt, inp)
    return out[0, 0]


def kernel(inp, target):
    n, v = inp.shape
    tgt = target.astype(jnp.int32)
    sc_part = _sc_gather(inp, tgt)
    lse_sum = _loss_call(inp, tgt.reshape(n, 1))
    return lse_sum - jnp.sum(sc_part)


# speedup vs baseline: 1.1085x; 1.0083x over previous
import functools

import jax
import jax.numpy as jnp
from jax import lax
from jax.experimental import pallas as pl
from jax.experimental.pallas import tpu as pltpu
from jax.experimental.pallas import tpu_sc as plsc

_N = 4096
_V = 32000
_LANES = 16
_NW = 32
_RPW = _N // _NW
_CHUNKS = _RPW // _LANES


@functools.partial(
    pl.kernel,
    mesh=plsc.VectorSubcoreMesh(core_axis_name="c", subcore_axis_name="s"),
    out_type=jax.ShapeDtypeStruct((_NW * _LANES,), jnp.float32),
    compiler_params=pltpu.CompilerParams(needs_layout_passes=False),
    scratch_types=[
        pltpu.VMEM((_RPW,), jnp.int32),
        pltpu.VMEM((_LANES, 8, 128), jnp.float32),
        pltpu.VMEM((_LANES,), jnp.float32),
        pltpu.SemaphoreType.DMA,
    ],
)
def _sc_gather(x_hbm, tgt_hbm, out_hbm, tgt_v, tiles_v, acc_v, sem):
    wid = lax.axis_index("s") * 2 + lax.axis_index("c")
    base = wid * _RPW
    pltpu.sync_copy(tgt_hbm.at[pl.ds(base, _RPW)], tgt_v)
    acc = jnp.zeros((_LANES,), jnp.float32)
    for c in range(_CHUNKS):
        tc_vec = tgt_v[pl.ds(c * _LANES, _LANES)]
        copies = []
        for k in range(_LANES):
            t = tc_vec[k]
            col0 = pl.multiple_of(lax.bitwise_and(t, jnp.int32(~127)), 128)
            r0 = base + c * _LANES + 8 * (k // 8)
            copies.append(
                pltpu.async_copy(
                    x_hbm.at[pl.ds(r0, 8), pl.ds(col0, 128)], tiles_v.at[k], sem
                )
            )
        for cp in copies:
            cp.wait()
        t = tgt_v[pl.ds(c * _LANES, _LANES)]
        ii = lax.iota(jnp.int32, _LANES)
        rowin = lax.bitwise_and(ii, 7)
        lane = lax.bitwise_and(t, 127)
        vals = plsc.load_gather(tiles_v, [ii, rowin, lane])
        acc = acc + jnp.where(t != 0, vals, 0.0)
    acc_v[...] = acc
    pltpu.sync_copy(acc_v, out_hbm.at[pl.ds(wid * _LANES, _LANES)])


def _loss_body(tgt_ref, x_ref, out_ref, m_ref, s_ref):
    r = pl.program_id(0)
    c = pl.program_id(1)
    x = x_ref[...]
    t = tgt_ref[...]
    R = x.shape[0]

    @pl.when(c == 0)
    def _():
        m_ref[...] = jnp.full((R, 1), -jnp.inf, jnp.float32)
        s_ref[...] = jnp.zeros((R, 1), jnp.float32)

    bm = jnp.max(x, axis=1, keepdims=True)
    m_old = m_ref[...]
    m_new = jnp.maximum(m_old, bm)
    s_ref[...] = s_ref[...] * jnp.exp(m_old - m_new) + jnp.sum(
        jnp.exp(x - m_new), axis=1, keepdims=True
    )
    m_ref[...] = m_new

    @pl.when(c == pl.num_programs(1) - 1)
    def _():
        lse = jnp.log(s_ref[...]) + m_ref[...]
        part = jnp.sum(jnp.where(t != 0, lse, 0.0), axis=(0, 1), keepdims=True)

        @pl.when(r == 0)
        def _():
            out_ref[...] = part

        @pl.when(r > 0)
        def _():
            out_ref[...] += part


@functools.partial(jax.jit, static_argnames=("row_block", "col_block", "interpret"))
def _loss_call(inp, tgt, row_block=128, col_block=32000, interpret=False):
    n, v = inp.shape
    out = pl.pallas_call(
        _loss_body,
        grid=(n // row_block, v // col_block),
        in_specs=[
            pl.BlockSpec((row_block, 1), lambda r, c: (r, 0)),
            pl.BlockSpec((row_block, col_block), lambda r, c: (r, c)),
        ],
        out_specs=pl.BlockSpec((1, 1), lambda r, c: (0, 0)),
        out_shape=jax.ShapeDtypeStruct((1, 1), jnp.float32),
        scratch_shapes=[
            pltpu.VMEM((row_block, 1), jnp.float32),
            pltpu.VMEM((row_block, 1), jnp.float32),
        ],
        interpret=interpret,
        compiler_params=pltpu.CompilerParams(vmem_limit_bytes=100 * 1024 * 1024),
    )(tgt, inp)
    return out[0, 0]


def kernel(inp, target):
    n, v = inp.shape
    tgt = target.astype(jnp.int32)
    sc_part = _sc_gather(inp, tgt)
    lse_sum = _loss_call(inp, tgt.reshape(n, 1))
    return lse_sum - jnp.sum(sc_part)
